# Initial kernel scaffold; baseline (speedup 1.0000x reference)
import jax
import jax.numpy as jnp
from jax.experimental import pallas as pl


def kernel(x, y, r, l, Wx, Wr):
    raise NotImplementedError("write your pallas kernel here")


# speedup vs baseline: 1.6659x; 1.6659x over previous
import functools

import jax
import jax.numpy as jnp
from jax import lax
from jax.experimental import pallas as pl
from jax.experimental.pallas import tpu as pltpu
from jax.experimental.pallas import tpu_sc as plsc

NUM_NODE = 10000
NUM_RELATION = 64
HIDDEN = 128
BATCH = 16384

NC, NS = 2, 16
NW = NC * NS
BPW = BATCH // NW
CH = 128
NCH = BPW // CH

_sc_mesh = plsc.VectorSubcoreMesh(core_axis_name="c", subcore_axis_name="s")


@functools.partial(
    pl.kernel,
    mesh=_sc_mesh,
    out_type=(
        jax.ShapeDtypeStruct((BATCH, HIDDEN), jnp.float32),
        jax.ShapeDtypeStruct((BATCH, HIDDEN), jnp.float32),
    ),
    scratch_types=[
        pltpu.VMEM((2, NCH, CH), jnp.int32),
        pltpu.VMEM((CH, HIDDEN), jnp.float32),
        pltpu.VMEM((CH, HIDDEN), jnp.float32),
        pltpu.SemaphoreType.DMA,
        pltpu.SemaphoreType.DMA,
    ],
)
def _sc_gather(idx_hbm, wx_hbm, outx_hbm, outy_hbm, idx_v, rows0, rows1, sem0, sem1):
    wid = lax.axis_index("s") * NC + lax.axis_index("c")
    base = wid * BPW
    pltpu.sync_copy(idx_hbm.at[wid], idx_v)

    rows = (rows0, rows1)
    sems = (sem0, sem1)
    outs = (outx_hbm, outy_hbm)
    chunks = [(t, c) for t in range(2) for c in range(NCH)]
    copies = {}
    t0, c0 = chunks[0]
    copies[0] = pltpu.async_copy(wx_hbm.at[idx_v.at[t0, c0]], rows[0], sems[0])
    for k, (t, c) in enumerate(chunks):
        if k + 1 < len(chunks):
            tn, cn = chunks[k + 1]
            b = (k + 1) % 2
            copies[k + 1] = pltpu.async_copy(
                wx_hbm.at[idx_v.at[tn, cn]], rows[b], sems[b]
            )
        copies[k].wait()
        pltpu.sync_copy(rows[k % 2], outs[t].at[pl.ds(base + c * CH, CH)])


_BB = 2048
_NB = BATCH // _BB


def _tc_body(embx_ref, emby_ref, r_ref, l_ref, wr_ref, logits_ref, loss_ref):
    i = pl.program_id(0)
    wr = wr_ref[...]
    s = 1.0 / (1.0 + jnp.exp(-jnp.clip(wr, -6.0, 6.0)))
    re_wr = s * (1.0 - s)
    r = r_ref[...]
    onehot = (r == lax.broadcasted_iota(jnp.int32, (_BB, NUM_RELATION), 1)).astype(
        jnp.float32
    )
    re = jnp.dot(onehot, re_wr, preferred_element_type=jnp.float32)
    dot = jnp.sum(embx_ref[...] * emby_ref[...] * re, axis=1, keepdims=True)
    p = 1.0 / (1.0 + jnp.exp(-dot))
    logits_ref[...] = jnp.concatenate([p, 1.0 - p], axis=1)
    lse = jnp.log(jnp.exp(p) + jnp.exp(1.0 - p))
    chosen = jnp.where(l_ref[...] == 0, p, 1.0 - p)
    part = jnp.sum(lse - chosen) * (1.0 / BATCH)

    @pl.when(i == 0)
    def _():
        loss_ref[...] = jnp.zeros_like(loss_ref)

    loss_ref[...] = loss_ref[...] + part.reshape(1, 1)


_tc_compute = pl.pallas_call(
    _tc_body,
    grid=(_NB,),
    in_specs=[
        pl.BlockSpec((_BB, HIDDEN), lambda i: (i, 0)),
        pl.BlockSpec((_BB, HIDDEN), lambda i: (i, 0)),
        pl.BlockSpec((_BB, 1), lambda i: (i, 0)),
        pl.BlockSpec((_BB, 1), lambda i: (i, 0)),
        pl.BlockSpec((NUM_RELATION, HIDDEN), lambda i: (0, 0)),
    ],
    out_specs=[
        pl.BlockSpec((_BB, 2), lambda i: (i, 0)),
        pl.BlockSpec((1, 1), lambda i: (0, 0)),
    ],
    out_shape=[
        jax.ShapeDtypeStruct((BATCH, 2), jnp.float32),
        jax.ShapeDtypeStruct((1, 1), jnp.float32),
    ],
)


def kernel(x, y, r, l, Wx, Wr):
    xi = x.astype(jnp.int32).reshape(NW, NCH, CH)
    yi = y.astype(jnp.int32).reshape(NW, NCH, CH)
    idx = jnp.stack([xi, yi], axis=1)
    embx, emby = _sc_gather(idx, Wx)
    logits, loss = _tc_compute(
        embx,
        emby,
        r.astype(jnp.int32).reshape(BATCH, 1),
        l.astype(jnp.int32).reshape(BATCH, 1),
        Wr,
    )
    return logits, loss.reshape(())
